# Initial kernel scaffold; baseline (speedup 1.0000x reference)
import functools

import jax
import jax.numpy as jnp
from jax import lax
from jax.experimental import pallas as pl
from jax.experimental.pallas import tpu as pltpu
from jax.experimental.pallas import tpu_sc as plsc

_NUM_EMB = 1000000
_D = 32
_BATCH = 4096
_HIST = 200
_B = _BATCH * _HIST
_NW = 32
_BPW = _B // _NW
_CH = 1600
_NCH = _BPW // _CH


def _emb_body(idx_hbm, table_hbm, out_hbm, idx_v, rows_v, sem):
    c = lax.axis_index("c")
    s = lax.axis_index("s")
    wid = s * 2 + c
    base = wid * _BPW

    def step(i, carry):
        off = base + i * _CH
        pltpu.sync_copy(idx_hbm.at[pl.ds(off, _CH)], idx_v)

        def addone(j, carry2):
            sl = pl.ds(j * 16, 16)
            idx_v[sl] = idx_v[sl] + 1
            return carry2

        lax.fori_loop(0, _CH // 16, addone, 0, unroll=8)
        pltpu.async_copy(table_hbm.at[idx_v], rows_v, sem).wait()
        pltpu.sync_copy(rows_v, out_hbm.at[pl.ds(off, _CH)])
        return carry

    lax.fori_loop(0, _NCH, step, 0)


@functools.partial(
    pl.kernel,
    out_type=jax.ShapeDtypeStruct((_B, _D), jnp.float32),
    mesh=plsc.VectorSubcoreMesh(core_axis_name="c", subcore_axis_name="s"),
    scratch_types=[
        pltpu.VMEM((_CH,), jnp.int32),
        pltpu.VMEM((_CH, _D), jnp.float32),
        pltpu.SemaphoreType.DMA,
    ],
)
def _emb(idx_hbm, table_hbm, out_hbm, idx_v, rows_v, sem):
    _emb_body(idx_hbm, table_hbm, out_hbm, idx_v, rows_v, sem)


def kernel(input, table):
    out = _emb(input.reshape(_B), table)
    return out.reshape(_BATCH, _HIST, _D)


# speedup vs baseline: 1.4764x; 1.4764x over previous
import functools

import jax
import jax.numpy as jnp
from jax import lax
from jax.experimental import pallas as pl
from jax.experimental.pallas import tpu as pltpu
from jax.experimental.pallas import tpu_sc as plsc

_NUM_EMB = 1000000
_D = 32
_BATCH = 4096
_HIST = 200
_B = _BATCH * _HIST
_NW = 32
_BPW = _B // _NW
_CH = 1600
_NCH = _BPW // _CH


def _emb_body(idx_hbm, table_hbm, out_hbm, idx_v, rows_v, sem):
    c = lax.axis_index("c")
    s = lax.axis_index("s")
    wid = s * 2 + c
    base = wid * _BPW

    def step(i, carry):
        off = base + i * _CH
        pltpu.sync_copy(idx_hbm.at[pl.ds(off, _CH)], idx_v)

        def addone(j, carry2):
            sl = pl.ds(j * 16, 16)
            idx_v[sl] = idx_v[sl] + 1
            return carry2

        lax.fori_loop(0, _CH // 16, addone, 0, unroll=8)
        pltpu.async_copy(table_hbm.at[idx_v], rows_v, sem).wait()
        pltpu.sync_copy(rows_v, out_hbm.at[pl.ds(off, _CH)])
        return carry

    lax.fori_loop(0, _NCH, step, 0)


@functools.partial(
    pl.kernel,
    out_type=jax.ShapeDtypeStruct((_B, _D), jnp.float32),
    mesh=plsc.VectorSubcoreMesh(core_axis_name="c", subcore_axis_name="s"),
    compiler_params=pltpu.CompilerParams(use_tc_tiling_on_sc=False),
    scratch_types=[
        pltpu.VMEM((_CH,), jnp.int32),
        pltpu.VMEM((_CH, _D), jnp.float32),
        pltpu.SemaphoreType.DMA,
    ],
)
def _emb(idx_hbm, table_hbm, out_hbm, idx_v, rows_v, sem):
    _emb_body(idx_hbm, table_hbm, out_hbm, idx_v, rows_v, sem)


def kernel(input, table):
    out = _emb(input.reshape(_B), table)
    return out.reshape(_BATCH, _HIST, _D)


# speedup vs baseline: 1.4941x; 1.0120x over previous
import functools

import jax
import jax.numpy as jnp
from jax import lax
from jax.experimental import pallas as pl
from jax.experimental.pallas import tpu as pltpu
from jax.experimental.pallas import tpu_sc as plsc

_NUM_EMB = 1000000
_D = 32
_BATCH = 4096
_HIST = 200
_B = _BATCH * _HIST
_NW = 32
_BPW = _B // _NW
_CH = 1600
_NCH = _BPW // _CH


def _emb_body(idx_hbm, table_hbm, out_hbm, idx_v, rows_v, sem_idx, sem_gat,
              sem_wb):
    c = lax.axis_index("c")
    s = lax.axis_index("s")
    wid = s * 2 + c
    base = wid * _BPW

    def idx_cp(i, b):
        return pltpu.make_async_copy(
            idx_hbm.at[pl.ds(base + i * _CH, _CH)], idx_v.at[b],
            sem_idx.at[b])

    def gat_cp(b):
        return pltpu.make_async_copy(
            table_hbm.at[idx_v.at[b]], rows_v.at[b], sem_gat.at[b])

    def wb_cp(i, b):
        return pltpu.make_async_copy(
            rows_v.at[b], out_hbm.at[pl.ds(base + i * _CH, _CH)],
            sem_wb.at[b])

    def addone(b):
        def body(j, carry):
            sl = pl.ds(j * 16, 16)
            idx_v[b, sl] = idx_v[b, sl] + 1
            return carry

        lax.fori_loop(0, _CH // 16, body, 0, unroll=8)

    idx_cp(0, 0).start()
    for i in range(_NCH):
        b = i & 1
        idx_cp(i, b).wait()
        addone(b)
        if i + 1 < _NCH:
            idx_cp(i + 1, b ^ 1).start()
        if i >= 2:
            wb_cp(i - 2, b).wait()
        gat_cp(b).start()
        gat_cp(b).wait()
        wb_cp(i, b).start()
    wb_cp(_NCH - 2, _NCH & 1).wait()
    wb_cp(_NCH - 1, (_NCH - 1) & 1).wait()


@functools.partial(
    pl.kernel,
    out_type=jax.ShapeDtypeStruct((_B, _D), jnp.float32),
    mesh=plsc.VectorSubcoreMesh(core_axis_name="c", subcore_axis_name="s"),
    compiler_params=pltpu.CompilerParams(use_tc_tiling_on_sc=False),
    scratch_types=[
        pltpu.VMEM((2, _CH), jnp.int32),
        pltpu.VMEM((2, _CH, _D), jnp.float32),
        pltpu.SemaphoreType.DMA((2,)),
        pltpu.SemaphoreType.DMA((2,)),
        pltpu.SemaphoreType.DMA((2,)),
    ],
)
def _emb(idx_hbm, table_hbm, out_hbm, idx_v, rows_v, sem_idx, sem_gat,
         sem_wb):
    _emb_body(idx_hbm, table_hbm, out_hbm, idx_v, rows_v, sem_idx, sem_gat,
              sem_wb)


def kernel(input, table):
    out = _emb(input.reshape(_B), table)
    return out.reshape(_BATCH, _HIST, _D)
